# baseline (speedup 1.0000x reference)
import jax
import jax.numpy as jnp
from jax import lax
from jax.experimental import pallas as pl
from jax.experimental.pallas import tpu as pltpu
from jax.experimental.pallas import tpu_sc as plsc

EMB_DIM = 64
BATCH = 16384
VOCAB = 100000

_info = plsc.get_sparse_core_info()
_NC, _NS, _L = _info.num_cores, _info.num_subcores, _info.num_lanes
_WAVES = EMB_DIM // (_NC * _NS)
_ORING = 2
_NBLK = 4
_BLK = BATCH // _NBLK
_UNROLL = 16


def _sc_body(idx_hbm, wrT_hbm, g_hbm, col_v, idx_v, ob_v, semi, semr, semw):
    s = lax.axis_index("c")
    t = lax.axis_index("s")

    idxcp = pltpu.async_copy(idx_hbm, idx_v, semi)
    wcps = []
    for wave in range(_WAVES):
        d = s * (_WAVES * _NS) + wave * _NS + t
        colcp = pltpu.async_copy(wrT_hbm.at[d], col_v, semr)
        if wave == 0:
            idxcp.wait()
        colcp.wait()
        for b in range(_NBLK):
            base = b * _BLK
            ob = (wave * _NBLK + b) % _ORING
            if len(wcps) >= _ORING:
                wcps.pop(0).wait()

            @plsc.parallel_loop(0, _BLK // _L, unroll=_UNROLL)
            def _(m):
                i16 = idx_v[pl.ds(base + m * _L, _L)]
                w16 = plsc.load_gather(col_v, [i16])
                ob_v[ob, pl.ds(m * _L, _L)] = w16 * w16

            wcps.append(pltpu.async_copy(
                ob_v.at[ob], g_hbm.at[d, pl.ds(base, _BLK)], semw))
    for cp in wcps:
        cp.wait()


def _tc_reduce(s_ref, g_ref, o_ref):
    o_ref[...] = jnp.sum(s_ref[...] * g_ref[...], axis=0)


def kernel(sbj_embs, obj_embs, rel_ids, W_r):
    mesh = plsc.VectorSubcoreMesh(core_axis_name="c", subcore_axis_name="s")
    k = pl.kernel(
        _sc_body,
        mesh=mesh,
        compiler_params=pltpu.CompilerParams(
            needs_layout_passes=False, use_tc_tiling_on_sc=True),
        out_type=jax.ShapeDtypeStruct((EMB_DIM, BATCH), jnp.float32),
        scratch_types=[
            pltpu.VMEM((VOCAB,), jnp.float32),
            pltpu.VMEM((BATCH,), jnp.int32),
            pltpu.VMEM((_ORING, _BLK), jnp.float32),
            pltpu.SemaphoreType.DMA,
            pltpu.SemaphoreType.DMA,
            pltpu.SemaphoreType.DMA,
        ],
    )
    g = k(rel_ids.astype(jnp.int32), W_r.T)

    nblk = 2
    blk = BATCH // nblk
    return pl.pallas_call(
        _tc_reduce,
        grid=(nblk,),
        in_specs=[
            pl.BlockSpec((EMB_DIM, blk), lambda i: (0, i)),
            pl.BlockSpec((EMB_DIM, blk), lambda i: (0, i)),
        ],
        out_specs=pl.BlockSpec((blk,), lambda i: (i,)),
        out_shape=jax.ShapeDtypeStruct((BATCH,), jnp.float32),
    )(sbj_embs.T, g)


# speedup vs baseline: 1.0049x; 1.0049x over previous
import jax
import jax.numpy as jnp
from jax import lax
from jax.experimental import pallas as pl
from jax.experimental.pallas import tpu as pltpu
from jax.experimental.pallas import tpu_sc as plsc

EMB_DIM = 64
BATCH = 16384
VOCAB = 100000

_info = plsc.get_sparse_core_info()
_NC, _NS, _L = _info.num_cores, _info.num_subcores, _info.num_lanes
_WAVES = EMB_DIM // (_NC * _NS)
_ORING = 2
_NBLK = 4
_BLK = BATCH // _NBLK
_UNROLL = 8


def _sc_body(idx_hbm, wrT_hbm, g_hbm, col_v, idx_v, ob_v, semi, semr, semw):
    s = lax.axis_index("c")
    t = lax.axis_index("s")

    idxcp = pltpu.async_copy(idx_hbm, idx_v, semi)
    wcps = []
    for wave in range(_WAVES):
        d = s * (_WAVES * _NS) + wave * _NS + t
        colcp = pltpu.async_copy(wrT_hbm.at[d], col_v, semr)
        if wave == 0:
            idxcp.wait()
        colcp.wait()
        for b in range(_NBLK):
            base = b * _BLK
            ob = (wave * _NBLK + b) % _ORING
            if len(wcps) >= _ORING:
                wcps.pop(0).wait()

            @plsc.parallel_loop(0, _BLK // _L, unroll=_UNROLL)
            def _(m):
                i16 = idx_v[pl.ds(base + m * _L, _L)]
                w16 = plsc.load_gather(col_v, [i16])
                ob_v[ob, pl.ds(m * _L, _L)] = w16 * w16

            wcps.append(pltpu.async_copy(
                ob_v.at[ob], g_hbm.at[d, pl.ds(base, _BLK)], semw))
    for cp in wcps:
        cp.wait()


def _tc_reduce(s_ref, g_ref, o_ref):
    o_ref[...] = jnp.sum(s_ref[...] * g_ref[...], axis=0)


def kernel(sbj_embs, obj_embs, rel_ids, W_r):
    mesh = plsc.VectorSubcoreMesh(core_axis_name="c", subcore_axis_name="s")
    k = pl.kernel(
        _sc_body,
        mesh=mesh,
        compiler_params=pltpu.CompilerParams(
            needs_layout_passes=False, use_tc_tiling_on_sc=True),
        out_type=jax.ShapeDtypeStruct((EMB_DIM, BATCH), jnp.float32),
        scratch_types=[
            pltpu.VMEM((VOCAB,), jnp.float32),
            pltpu.VMEM((BATCH,), jnp.int32),
            pltpu.VMEM((_ORING, _BLK), jnp.float32),
            pltpu.SemaphoreType.DMA,
            pltpu.SemaphoreType.DMA,
            pltpu.SemaphoreType.DMA,
        ],
    )
    g = k(rel_ids.astype(jnp.int32), W_r.T)

    nblk = 1
    blk = BATCH // nblk
    return pl.pallas_call(
        _tc_reduce,
        grid=(nblk,),
        in_specs=[
            pl.BlockSpec((EMB_DIM, blk), lambda i: (0, i)),
            pl.BlockSpec((EMB_DIM, blk), lambda i: (0, i)),
        ],
        out_specs=pl.BlockSpec((blk,), lambda i: (i,)),
        out_shape=jax.ShapeDtypeStruct((BATCH,), jnp.float32),
    )(sbj_embs.T, g)


# speedup vs baseline: 1.0205x; 1.0155x over previous
import jax
import jax.numpy as jnp
from jax import lax
from jax.experimental import pallas as pl
from jax.experimental.pallas import tpu as pltpu
from jax.experimental.pallas import tpu_sc as plsc

EMB_DIM = 64
BATCH = 16384
VOCAB = 100000

_info = plsc.get_sparse_core_info()
_NC, _NS, _L = _info.num_cores, _info.num_subcores, _info.num_lanes
_WAVES = EMB_DIM // (_NC * _NS)
_ORING = 2
_NBLK = 4
_BLK = BATCH // _NBLK
_UNROLL = 8


def _sc_body(idx_hbm, wrT_hbm, g_hbm, col_v, idx_v, ob_v, semi, semr, semw):
    s = lax.axis_index("c")
    t = lax.axis_index("s")

    idxcp = pltpu.async_copy(idx_hbm, idx_v, semi)
    wcps = []
    for wave in range(_WAVES):
        d = s * (_WAVES * _NS) + wave * _NS + t
        colcp = pltpu.async_copy(wrT_hbm.at[d], col_v, semr)
        if wave == 0:
            idxcp.wait()
        colcp.wait()
        for b in range(_NBLK):
            base = b * _BLK
            ob = (wave * _NBLK + b) % _ORING
            if len(wcps) >= _ORING:
                wcps.pop(0).wait()

            @plsc.parallel_loop(0, _BLK // _L, unroll=_UNROLL)
            def _(m):
                i16 = idx_v[pl.ds(base + m * _L, _L)]
                w16 = plsc.load_gather(col_v, [i16])
                ob_v[ob, pl.ds(m * _L, _L)] = w16 * w16

            wcps.append(pltpu.async_copy(
                ob_v.at[ob], g_hbm.at[d, pl.ds(base, _BLK)], semw))
    for cp in wcps:
        cp.wait()


def _tc_reduce(s_ref, g_ref, o_ref):
    o_ref[...] = jnp.sum(s_ref[...] * g_ref[...], axis=0)


def kernel(sbj_embs, obj_embs, rel_ids, W_r):
    mesh = plsc.VectorSubcoreMesh(core_axis_name="c", subcore_axis_name="s")
    k = pl.kernel(
        _sc_body,
        mesh=mesh,
        compiler_params=pltpu.CompilerParams(
            needs_layout_passes=False, use_tc_tiling_on_sc=True),
        out_type=jax.ShapeDtypeStruct((EMB_DIM, BATCH), jnp.float32),
        scratch_types=[
            pltpu.VMEM((VOCAB,), jnp.float32),
            pltpu.VMEM((BATCH,), jnp.int32),
            pltpu.VMEM((_ORING, _BLK), jnp.float32),
            pltpu.SemaphoreType.DMA,
            pltpu.SemaphoreType.DMA,
            pltpu.SemaphoreType.DMA,
        ],
    )
    g = k(rel_ids.astype(jnp.int32), W_r.T)

    nblk = 2
    blk = BATCH // nblk
    return pl.pallas_call(
        _tc_reduce,
        grid=(nblk,),
        in_specs=[
            pl.BlockSpec((EMB_DIM, blk), lambda i: (0, i)),
            pl.BlockSpec((EMB_DIM, blk), lambda i: (0, i)),
        ],
        out_specs=pl.BlockSpec((blk,), lambda i: (i,)),
        out_shape=jax.ShapeDtypeStruct((BATCH,), jnp.float32),
    )(sbj_embs.T, g)
